# baseline (speedup 1.0000x reference)
import jax
import jax.numpy as jnp
from jax import lax
from jax.experimental import pallas as pl
from jax.experimental.pallas import tpu as pltpu
from jax.experimental.pallas import tpu_sc as plsc

_B = 4096
_N = 64
_D = 64
_L = 16
_NC = 2
_NS = 16
_NW = _NC * _NS
_BW = _B // _NW
_RPW = 2 * _N * _BW
_CH = 128
_NCHUNK = _RPW // _CH
_NBUF = 4
_MARGIN = 9.0
_K = _D // _L


def _sc_body(head_h, rel_h, tail_h, negidx_h, ent_h, relemb_h,
             pos_out, neg_out,
             hidx, ridx, tidx, idx_v, c_v, hrow, rrow, trow,
             buf0, buf1, buf2, buf3, score_v, pos_v,
             sem_a, sem0, sem1, sem2, sem3):
  bufs = (buf0, buf1, buf2, buf3)
  sems = (sem0, sem1, sem2, sem3)
  wid = lax.axis_index("s") * _NC + lax.axis_index("c")
  base = wid * _BW
  nbase = wid * (_BW * _N)
  lane = lax.iota(jnp.int32, _L)

  pltpu.sync_copy(head_h.at[pl.ds(base, _BW)], hidx)
  pltpu.sync_copy(rel_h.at[pl.ds(base, _BW)], ridx)
  pltpu.sync_copy(tail_h.at[pl.ds(base, _BW)], tidx)
  pltpu.async_copy(ent_h.at[hidx], hrow, sem_a).wait()
  pltpu.async_copy(relemb_h.at[ridx], rrow, sem_a).wait()
  pltpu.async_copy(ent_h.at[tidx], trow, sem_a).wait()
  pltpu.sync_copy(negidx_h.at[pl.ds(nbase, _BW * _N)],
                  idx_v.at[pl.ds(0, _BW * _N)])
  pltpu.sync_copy(negidx_h.at[pl.ds(_B * _N + nbase, _BW * _N)],
                  idx_v.at[pl.ds(_BW * _N, _BW * _N)])

  def _load_row(ref, r):
    return [ref[r, pl.ds(k * _L, _L)] for k in range(_K)]

  def _lane_total(sq):
    return jnp.sum((sq[0] + sq[1]) + (sq[2] + sq[3]))

  def p1_grp(g, carry):
    s_vec = jnp.zeros((_L,), jnp.float32)
    for j in range(_L):
      b = g * _L + j
      h_k = _load_row(hrow, b)
      r_k = _load_row(rrow, b)
      t_k = _load_row(trow, b)
      sq = []
      for k in range(_K):
        ch = r_k[k] - t_k[k]
        ct = -(h_k[k] + r_k[k])
        c_v[b, pl.ds(k * _L, _L)] = ch
        c_v[_BW + b, pl.ds(k * _L, _L)] = ct
        d = h_k[k] + ch
        sq.append(d * d)
      s_vec = jnp.where(lane == j, _lane_total(sq), s_vec)
    pos_v[pl.ds(g * _L, _L)] = s_vec
    return carry
  lax.fori_loop(0, _BW // _L, p1_grp, 0)
  pltpu.sync_copy(pos_v, pos_out.at[pl.ds(base, _BW)])

  def start(cidx, buf, sem):
    pltpu.async_copy(ent_h.at[idx_v.at[pl.ds(cidx * _CH, _CH)]], buf, sem)

  def wait(buf, sem):
    pltpu.make_async_copy(ent_h.at[idx_v.at[pl.ds(0, _CH)]], buf, sem).wait()

  for r in range(_NBUF - 1):
    start(r, bufs[r], sems[r])

  def chunk_body(cidx, buf):
    def grp(gg, carry):
      bl = 2 * cidx + gg // (_N // _L)
      c_k = _load_row(c_v, bl)
      s_vec = jnp.zeros((_L,), jnp.float32)
      for j in range(_L):
        row = gg * _L + j
        x_k = _load_row(buf, row)
        sq = []
        for k in range(_K):
          t = x_k[k] + c_k[k]
          sq.append(t * t)
        s_vec = jnp.where(lane == j, _lane_total(sq), s_vec)
      score_v[pl.ds(cidx * _CH + gg * _L, _L)] = s_vec
      return carry
    lax.fori_loop(0, _CH // _L, grp, 0)

  def ring_iter(i, carry):
    for r in range(_NBUF):
      cidx = i * _NBUF + r
      wait(bufs[r], sems[r])
      chunk_body(cidx, bufs[r])
      nxt = cidx + (_NBUF - 1)
      @pl.when(nxt < _NCHUNK)
      def _():
        start(nxt, bufs[(r + _NBUF - 1) % _NBUF],
              sems[(r + _NBUF - 1) % _NBUF])
    return carry
  lax.fori_loop(0, _NCHUNK // _NBUF, ring_iter, 0)

  pltpu.sync_copy(score_v.at[pl.ds(0, _BW * _N)],
                  neg_out.at[pl.ds(nbase, _BW * _N)])
  pltpu.sync_copy(score_v.at[pl.ds(_BW * _N, _BW * _N)],
                  neg_out.at[pl.ds(_B * _N + nbase, _BW * _N)])


def _sc_scores(head, rel, tail, neg_idx, ent_emb, rel_emb):
  mesh = plsc.VectorSubcoreMesh(core_axis_name="c", subcore_axis_name="s",
                                num_cores=_NC, num_subcores=_NS)
  return pl.kernel(
      _sc_body,
      out_type=(jax.ShapeDtypeStruct((_B,), jnp.float32),
                jax.ShapeDtypeStruct((2 * _B * _N,), jnp.float32)),
      mesh=mesh,
      compiler_params=pltpu.CompilerParams(needs_layout_passes=False,
                                           use_tc_tiling_on_sc=False),
      scratch_types=[
          pltpu.VMEM((_BW,), jnp.int32),
          pltpu.VMEM((_BW,), jnp.int32),
          pltpu.VMEM((_BW,), jnp.int32),
          pltpu.VMEM((_RPW,), jnp.int32),
          pltpu.VMEM((2 * _BW, _D), jnp.float32),
          pltpu.VMEM((_BW, _D), jnp.float32),
          pltpu.VMEM((_BW, _D), jnp.float32),
          pltpu.VMEM((_BW, _D), jnp.float32),
          pltpu.VMEM((_CH, _D), jnp.float32),
          pltpu.VMEM((_CH, _D), jnp.float32),
          pltpu.VMEM((_CH, _D), jnp.float32),
          pltpu.VMEM((_CH, _D), jnp.float32),
          pltpu.VMEM((_RPW,), jnp.float32),
          pltpu.VMEM((_BW,), jnp.float32),
          pltpu.SemaphoreType.DMA,
          pltpu.SemaphoreType.DMA,
          pltpu.SemaphoreType.DMA,
          pltpu.SemaphoreType.DMA,
          pltpu.SemaphoreType.DMA,
      ],
  )(head, rel, tail, neg_idx, ent_emb, rel_emb)


def _tc_loss_body(pos_ref, neg_ref, out_ref):
  def sp(x):
    return jnp.maximum(x, 0.0) + jnp.log1p(jnp.exp(-jnp.abs(x)))
  pos_score = _MARGIN - jnp.sqrt(pos_ref[...] + 1e-12)
  neg_score = _MARGIN - jnp.sqrt(neg_ref[...] + 1e-12)
  out_ref[0, 0] = jnp.mean(sp(-pos_score)) + jnp.mean(sp(neg_score))


def _tc_loss(pos2d, neg2d):
  return pl.pallas_call(
      _tc_loss_body,
      out_specs=pl.BlockSpec(memory_space=pltpu.SMEM),
      out_shape=jax.ShapeDtypeStruct((1, 1), jnp.float32),
  )(pos2d, neg2d)


def kernel(head, rel, tail, head_negs, tail_negs, ent_emb, rel_emb):
  head = head.astype(jnp.int32)
  rel = rel.astype(jnp.int32)
  tail = tail.astype(jnp.int32)
  neg_idx = jnp.concatenate([head_negs.reshape(-1),
                             tail_negs.reshape(-1)]).astype(jnp.int32)
  pos_sq, neg_sq = _sc_scores(head, rel, tail, neg_idx, ent_emb, rel_emb)
  loss = _tc_loss(pos_sq.reshape(_NW, _BW),
                  neg_sq.reshape(2 * _B * _N // 128, 128))
  return loss[0, 0]


# speedup vs baseline: 1.3459x; 1.3459x over previous
import jax
import jax.numpy as jnp
from jax import lax
from jax.experimental import pallas as pl
from jax.experimental.pallas import tpu as pltpu
from jax.experimental.pallas import tpu_sc as plsc

_B = 4096
_N = 64
_D = 64
_L = 16
_NC = 2
_NS = 16
_NW = _NC * _NS
_BW = _B // _NW
_RPW = 2 * _N * _BW
_CH = 64
_NCHUNK = _RPW // _CH
_NBUF = 4
_MARGIN = 9.0
_K = _D // _L


def _row_dmas(table_h, idx_ref, buf, sem):
  def issue(g, carry):
    vec = idx_ref[pl.ds(g * _L, _L)]
    for j in range(_L):
      pltpu.async_copy(table_h.at[vec[j]], buf.at[g * _L + j], sem)
    return carry
  lax.fori_loop(0, buf.shape[0] // _L, issue, 0)


def _drain(dummy_src, buf, sem):
  pltpu.make_async_copy(dummy_src, buf, sem).wait()


def _sc_body(head_h, rel_h, tail_h, negidx_h, ent_h, relemb_h,
             pos_out, neg_out,
             c_v, buf0, buf1, buf2, buf3, scr0, scr1, scr2, scr3, pos_v,
             hrt_v, iv0, iv1, iv2, iv3,
             sem_a, wsem, semi0, semi1, semi2, semi3,
             sem0, sem1, sem2, sem3):
  bufs = (buf0, buf1, buf2, buf3)
  scrs = (scr0, scr1, scr2, scr3)
  sems = (sem0, sem1, sem2, sem3)
  semis = (semi0, semi1, semi2, semi3)
  idx_v = (iv0, iv1, iv2, iv3)
  wid = lax.axis_index("s") * _NC + lax.axis_index("c")
  base = wid * _BW
  nbase = wid * (_BW * _N)
  lane = lax.iota(jnp.int32, _L)
  dummy = ent_h.at[pl.ds(0, _CH), :]
  idummy = negidx_h.at[pl.ds(0, _CH)]
  wdummy = neg_out.at[pl.ds(0, _CH)]

  def _load_row(ref, r):
    return [ref[r, pl.ds(k * _L, _L)] for k in range(_K)]

  def _lane_totals(ps):
    s_vec = jnp.zeros((_L,), jnp.float32)
    for j, p in enumerate(ps):
      s_vec = jnp.where(lane == j, jnp.sum(p), s_vec)
    return s_vec

  pltpu.sync_copy(head_h.at[pl.ds(base, _BW)], hrt_v.at[pl.ds(0, _BW)])
  pltpu.sync_copy(rel_h.at[pl.ds(base, _BW)], hrt_v.at[pl.ds(_BW, _BW)])
  pltpu.sync_copy(tail_h.at[pl.ds(base, _BW)], hrt_v.at[pl.ds(2 * _BW, _BW)])
  for half in range(2):
    hs = half * _CH
    def issue3(g, carry):
      hv = hrt_v[pl.ds(hs + g * _L, _L)]
      rv = hrt_v[pl.ds(_BW + hs + g * _L, _L)]
      tv = hrt_v[pl.ds(2 * _BW + hs + g * _L, _L)]
      for j in range(_L):
        row = g * _L + j
        pltpu.async_copy(ent_h.at[hv[j]], buf0.at[row], sem_a)
        pltpu.async_copy(relemb_h.at[rv[j]], buf1.at[row], sem_a)
        pltpu.async_copy(ent_h.at[tv[j]], buf2.at[row], sem_a)
      return carry
    lax.fori_loop(0, _CH // _L, issue3, 0)
    _drain(dummy, buf0, sem_a)
    _drain(dummy, buf1, sem_a)
    _drain(dummy, buf2, sem_a)
    for g in range(_CH // _L):
      ps = []
      for j in range(_L):
        b = half * _CH + g * _L + j
        r_loc = g * _L + j
        h_k = _load_row(buf0, r_loc)
        r_k = _load_row(buf1, r_loc)
        t_k = _load_row(buf2, r_loc)
        p = None
        for k in range(_K):
          ch = r_k[k] - t_k[k]
          ct = -(h_k[k] + r_k[k])
          c_v[b, pl.ds(k * _L, _L)] = ch
          c_v[_BW + b, pl.ds(k * _L, _L)] = ct
          d = h_k[k] + ch
          sq = d * d
          p = sq if p is None else p + sq
        ps.append(p)
      pos_v[pl.ds(half * _CH + g * _L, _L)] = _lane_totals(ps)
  pltpu.sync_copy(pos_v, pos_out.at[pl.ds(base, _BW)])

  def in_off(c):
    return jnp.where(c < _NCHUNK // 2,
                     nbase + c * _CH,
                     _B * _N + nbase + (c - _NCHUNK // 2) * _CH)

  def idx_prefetch(c, r):
    pltpu.async_copy(negidx_h.at[pl.ds(in_off(c), _CH)], idx_v[r], semis[r])

  def rows_start(c, r):
    _drain(idummy, idx_v[r], semis[r])
    _row_dmas(ent_h, idx_v[r], bufs[r], sems[r])

  for r in range(_NBUF):
    idx_prefetch(r, r)
  for r in range(_NBUF - 1):
    rows_start(r, r)

  def chunk_body(cidx, buf, scr):
    def grp(gg, carry):
      c_k = _load_row(c_v, cidx)
      ps = []
      for j in range(_L):
        row = gg * _L + j
        x_k = _load_row(buf, row)
        p = None
        for k in range(_K):
          t = x_k[k] + c_k[k]
          sq = t * t
          p = sq if p is None else p + sq
        ps.append(p)
      scr[pl.ds(gg * _L, _L)] = _lane_totals(ps)
      return carry
    lax.fori_loop(0, _CH // _L, grp, 0)

  def ring_iter(i, carry):
    for r in range(_NBUF):
      cidx = i * _NBUF + r
      _drain(dummy, bufs[r], sems[r])

      @pl.when(cidx >= _NBUF)
      def _():
        _drain(wdummy, scrs[r], wsem)
      chunk_body(cidx, bufs[r], scrs[r])
      pltpu.async_copy(scrs[r], neg_out.at[pl.ds(in_off(cidx), _CH)], wsem)

      nxt = cidx + (_NBUF - 1)
      @pl.when(nxt < _NCHUNK)
      def _():
        rn = (r + _NBUF - 1) % _NBUF
        rows_start(nxt, rn)
        @pl.when(nxt + 1 < _NCHUNK)
        def _():
          idx_prefetch(nxt + 1, r)
    return carry
  lax.fori_loop(0, _NCHUNK // _NBUF, ring_iter, 0)
  for r in range(_NBUF):
    _drain(wdummy, scrs[r], wsem)


def _sc_scores(head, rel, tail, neg_idx, ent_emb, rel_emb):
  mesh = plsc.VectorSubcoreMesh(core_axis_name="c", subcore_axis_name="s",
                                num_cores=_NC, num_subcores=_NS)
  return pl.kernel(
      _sc_body,
      out_type=(jax.ShapeDtypeStruct((_B,), jnp.float32),
                jax.ShapeDtypeStruct((2 * _B * _N,), jnp.float32)),
      mesh=mesh,
      compiler_params=pltpu.CompilerParams(needs_layout_passes=False),
      scratch_types=[
          pltpu.VMEM((2 * _BW, _D), jnp.float32),
          pltpu.VMEM((_CH, _D), jnp.float32),
          pltpu.VMEM((_CH, _D), jnp.float32),
          pltpu.VMEM((_CH, _D), jnp.float32),
          pltpu.VMEM((_CH, _D), jnp.float32),
          pltpu.VMEM((_CH,), jnp.float32),
          pltpu.VMEM((_CH,), jnp.float32),
          pltpu.VMEM((_CH,), jnp.float32),
          pltpu.VMEM((_CH,), jnp.float32),
          pltpu.VMEM((_BW,), jnp.float32),
          pltpu.VMEM((3 * _BW,), jnp.int32),
          pltpu.VMEM((_CH,), jnp.int32),
          pltpu.VMEM((_CH,), jnp.int32),
          pltpu.VMEM((_CH,), jnp.int32),
          pltpu.VMEM((_CH,), jnp.int32),
          pltpu.SemaphoreType.DMA,
          pltpu.SemaphoreType.DMA,
          pltpu.SemaphoreType.DMA,
          pltpu.SemaphoreType.DMA,
          pltpu.SemaphoreType.DMA,
          pltpu.SemaphoreType.DMA,
          pltpu.SemaphoreType.DMA,
          pltpu.SemaphoreType.DMA,
          pltpu.SemaphoreType.DMA,
          pltpu.SemaphoreType.DMA,
      ],
  )(head, rel, tail, neg_idx, ent_emb, rel_emb)


def _tc_loss_body(pos_ref, neg_ref, out_ref):
  def sp(x):
    return jnp.maximum(x, 0.0) + jnp.log1p(jnp.exp(-jnp.abs(x)))
  pos_score = _MARGIN - jnp.sqrt(pos_ref[...] + 1e-12)
  neg_score = _MARGIN - jnp.sqrt(neg_ref[...] + 1e-12)
  out_ref[0, 0] = jnp.mean(sp(-pos_score)) + jnp.mean(sp(neg_score))


def _tc_loss(pos2d, neg2d):
  return pl.pallas_call(
      _tc_loss_body,
      out_specs=pl.BlockSpec(memory_space=pltpu.SMEM),
      out_shape=jax.ShapeDtypeStruct((1, 1), jnp.float32),
  )(pos2d, neg2d)


def kernel(head, rel, tail, head_negs, tail_negs, ent_emb, rel_emb):
  head = head.astype(jnp.int32)
  rel = rel.astype(jnp.int32)
  tail = tail.astype(jnp.int32)
  neg_idx = jnp.concatenate([head_negs.reshape(-1),
                             tail_negs.reshape(-1)]).astype(jnp.int32)
  pos_sq, neg_sq = _sc_scores(head, rel, tail, neg_idx, ent_emb, rel_emb)
  loss = _tc_loss(pos_sq.reshape(_NW, _BW),
                  neg_sq.reshape(2 * _B * _N // 128, 128))
  return loss[0, 0]
